# Initial kernel scaffold; baseline (speedup 1.0000x reference)
import jax
import jax.numpy as jnp
from jax.experimental import pallas as pl


def kernel(x, edge_index, batch, smile_feature, target, gat_W, gat_as, gat_ad, gat_b, gcn1_W, gcn1_b, fcg11_W, fcg11_b, fcg21_W, fcg21_b, gcn2_W, gcn2_b, fcg12_W, fcg12_b, fcg22_W, fcg22_b, gcn3_W, gcn3_b, fcg13_W, fcg13_b, fcg23_W, fcg23_b, fus_W, fus_b, w3, emb, blf_Wih, blf_Whh, blf_bih, blf_bhh, blr_Wih, blr_Whh, blr_bih, blr_bhh, l_Wih, l_Whh, l_bih, l_bhh, fc1_W, fc1_b, fc2_W, fc2_b, fc3_W, fc3_b, fc4_W, fc4_b, out_W, out_b):
    raise NotImplementedError("write your pallas kernel here")


# speedup vs baseline: 1.9042x; 1.9042x over previous
import functools

import numpy as np
import jax
import jax.numpy as jnp
from jax import lax
from jax.experimental import pallas as pl
from jax.experimental.pallas import tpu as pltpu

N_NODES, N_EDGES, B_GRAPHS, SEQ_L = 10000, 160000, 128, 100
HEADS, DF = 10, 78
HP = 80
F = HEADS * HP
N_PAD = 10240
R_PAD = 104

_POS780 = np.arange(780)
_POS780 = (_POS780 // DF) * HP + (_POS780 % DF)


def _pool_mat_np(in_w=128, out_w=78):
    P = np.zeros((in_w, out_w), np.float32)
    for i in range(out_w):
        s = (i * in_w) // out_w
        e = int(np.ceil((i + 1) * in_w / out_w))
        P[s:e, i] = 1.0 / (e - s)
    return P


_POOL_NP = _pool_mat_np(128, 78)


def _pad_cols(W):
    out = jnp.zeros(W.shape[:-1] + (F,), jnp.float32)
    return out.at[..., _POS780].set(W)


def _pad_rows(W):
    out = jnp.zeros((F,) + W.shape[1:], jnp.float32)
    return out.at[_POS780].set(W)


def _pad_both(W):
    return _pad_cols(_pad_rows(W))


def _mm_kernel(a_ref, b_ref, bias_ref, s_ref, o_ref, *, act, scale):
    acc = jnp.dot(a_ref[...], b_ref[...], preferred_element_type=jnp.float32)
    acc = acc + bias_ref[...]
    if scale:
        acc = acc * s_ref[...]
    if act == "relu":
        acc = jnp.maximum(acc, 0.0)
    o_ref[...] = acc


def _mm(A, B, bias=None, act=None, row_scale=None, bm=1024):
    M, K = A.shape
    N = B.shape[1]
    Mp = ((M + bm - 1) // bm) * bm
    if Mp != M:
        A = jnp.pad(A, ((0, Mp - M), (0, 0)))
    if bias is None:
        bias = jnp.zeros((N,), jnp.float32)
    bias2 = bias.reshape(1, N)
    scale = row_scale is not None
    if scale:
        s = row_scale.reshape(M, 1)
        if Mp != M:
            s = jnp.pad(s, ((0, Mp - M), (0, 0)))
    else:
        s = jnp.zeros((Mp, 1), jnp.float32)
    out = pl.pallas_call(
        functools.partial(_mm_kernel, act=act, scale=scale),
        grid=(Mp // bm,),
        in_specs=[
            pl.BlockSpec((bm, K), lambda i: (i, 0)),
            pl.BlockSpec((K, N), lambda i: (0, 0)),
            pl.BlockSpec((1, N), lambda i: (0, 0)),
            pl.BlockSpec((bm, 1), lambda i: (i, 0)),
        ],
        out_specs=pl.BlockSpec((bm, N), lambda i: (i, 0)),
        out_shape=jax.ShapeDtypeStruct((Mp, N), jnp.float32),
    )(A, B, bias2, s)
    return out[:M] if Mp != M else out


def _gat_finish_kernel(agg_ref, den_ref, b_ref, o_ref):
    den = den_ref[...]
    acc = agg_ref[...]
    parts = []
    for h in range(HEADS):
        d = den[:, h * HP:h * HP + 1]
        parts.append(acc[:, h * HP:(h + 1) * HP] / (d + 1e-16))
    o_ref[...] = jnp.maximum(jnp.concatenate(parts, axis=1) + b_ref[...], 0.0)


def _gat_finish(agg, den_rep, bias, bm=1024):
    M = agg.shape[0]
    return pl.pallas_call(
        _gat_finish_kernel,
        grid=(M // bm,),
        in_specs=[
            pl.BlockSpec((bm, F), lambda i: (i, 0)),
            pl.BlockSpec((bm, F), lambda i: (i, 0)),
            pl.BlockSpec((1, F), lambda i: (0, 0)),
        ],
        out_specs=pl.BlockSpec((bm, F), lambda i: (i, 0)),
        out_shape=jax.ShapeDtypeStruct((M, F), jnp.float32),
    )(agg, den_rep, bias.reshape(1, F))


def _scale_bias_relu_add_kernel(a_ref, s_ref, b_ref, c_ref, o_ref, *, resid):
    v = jnp.maximum(a_ref[...] * s_ref[...] + b_ref[...], 0.0)
    if resid:
        v = v + c_ref[...]
    o_ref[...] = v


def _scale_bias_relu_add(a, row_scale, bias, resid=None, bm=1024):
    M, N = a.shape
    s = row_scale.reshape(M, 1)
    has_resid = resid is not None
    c = resid if has_resid else jnp.zeros((M, 1), jnp.float32)
    cN = c.shape[1]
    return pl.pallas_call(
        functools.partial(_scale_bias_relu_add_kernel, resid=has_resid),
        grid=(M // bm,),
        in_specs=[
            pl.BlockSpec((bm, N), lambda i: (i, 0)),
            pl.BlockSpec((bm, 1), lambda i: (i, 0)),
            pl.BlockSpec((1, N), lambda i: (0, 0)),
            pl.BlockSpec((bm, cN), lambda i: (i, 0)),
        ],
        out_specs=pl.BlockSpec((bm, N), lambda i: (i, 0)),
        out_shape=jax.ShapeDtypeStruct((M, N), jnp.float32),
    )(a, s, bias.reshape(1, N), c)


def _emb_kernel(idx_ref, emb_ref, o_ref):
    idx = idx_ref[...]
    iot = lax.broadcasted_iota(jnp.int32, (1, emb_ref.shape[0]), 1)
    onehot = (idx == iot).astype(jnp.float32)
    o_ref[...] = jnp.dot(onehot, emb_ref[...], preferred_element_type=jnp.float32)


def _emb_lookup(idx_flat, emb, bm=1024):
    M = idx_flat.shape[0]
    V, D = emb.shape
    return pl.pallas_call(
        _emb_kernel,
        grid=(M // bm,),
        in_specs=[
            pl.BlockSpec((bm, 1), lambda i: (i, 0)),
            pl.BlockSpec((V, D), lambda i: (0, 0)),
        ],
        out_specs=pl.BlockSpec((bm, D), lambda i: (i, 0)),
        out_shape=jax.ShapeDtypeStruct((M, D), jnp.float32),
    )(idx_flat.reshape(M, 1).astype(jnp.int32), emb)


def _lstm_rec_kernel(gx_ref, whh_ref, o_ref, h_ref, c_ref, *, Hg, mask_rows,
                     dual_half):
    t = pl.program_id(0)

    @pl.when(t == 0)
    def _():
        h_ref[...] = jnp.zeros_like(h_ref)
        c_ref[...] = jnp.zeros_like(c_ref)

    g = gx_ref[0] + jnp.dot(h_ref[...], whh_ref[...],
                            preferred_element_type=jnp.float32)
    i = g[:, 0 * Hg:1 * Hg]
    f = g[:, 1 * Hg:2 * Hg]
    gg = g[:, 2 * Hg:3 * Hg]
    o = g[:, 3 * Hg:4 * Hg]
    c = jax.nn.sigmoid(f) * c_ref[...] + jax.nn.sigmoid(i) * jnp.tanh(gg)
    h = jax.nn.sigmoid(o) * jnp.tanh(c)
    c_ref[...] = c
    if dual_half is None:
        h_ref[...] = h
    else:
        rows = lax.broadcasted_iota(jnp.int32, h.shape, 0)
        m = (rows < dual_half).astype(jnp.float32)
        h_ref[...] = jnp.concatenate([h * m, h * (1.0 - m)], axis=1)
    if mask_rows is not None:
        rows = lax.broadcasted_iota(jnp.int32, h.shape, 0)
        h = jnp.where(rows < mask_rows, h, 0.0)
    o_ref[0] = h


def _lstm_rec(gx, whh_t, Hg, mask_rows=None, dual_half=None):
    T, R, G = gx.shape
    Hw = whh_t.shape[0]
    return pl.pallas_call(
        functools.partial(_lstm_rec_kernel, Hg=Hg, mask_rows=mask_rows,
                          dual_half=dual_half),
        grid=(T,),
        in_specs=[
            pl.BlockSpec((1, R, G), lambda t: (t, 0, 0)),
            pl.BlockSpec((Hw, G), lambda t: (0, 0)),
        ],
        out_specs=pl.BlockSpec((1, R, Hg), lambda t: (t, 0, 0)),
        out_shape=jax.ShapeDtypeStruct((T, R, Hg), jnp.float32),
        scratch_shapes=[
            pltpu.VMEM((R, Hw), jnp.float32),
            pltpu.VMEM((R, Hg), jnp.float32),
        ],
    )(gx, whh_t)


def _gate_pad(Wih, Whh, bih, bhh, Hg):
    H4, K = Wih.shape
    H = H4 // 4
    Hin = Whh.shape[1]
    wih = jnp.zeros((K, 4 * Hg), jnp.float32)
    whh = jnp.zeros((Hg, 4 * Hg), jnp.float32)
    b = jnp.zeros((4 * Hg,), jnp.float32)
    for g in range(4):
        wih = wih.at[:, g * Hg:g * Hg + H].set(Wih[g * H:(g + 1) * H].T)
        whh = whh.at[:Hin, g * Hg:g * Hg + H].set(Whh[g * H:(g + 1) * H].T)
        b = b.at[g * Hg:g * Hg + H].set(bih[g * H:(g + 1) * H] +
                                        bhh[g * H:(g + 1) * H])
    return wih, whh, b


def _tail_kernel(lo_ref, pool_ref, p1_ref, p2_ref, p3_ref, w3_ref, tgt_ref,
                 w1p_ref, w1x_ref, w1t_ref, b1_ref, w2_ref, b2_ref,
                 w3w_ref, b3_ref, w4_ref, b4_ref, wo_ref, bo_ref, o_ref):
    losum = jnp.sum(lo_ref[...], axis=1) * (1.0 / SEQ_L)
    xd = jnp.dot(losum, pool_ref[...], preferred_element_type=jnp.float32)
    e0 = jnp.exp(w3_ref[0])
    e1 = jnp.exp(w3_ref[1])
    e2 = jnp.exp(w3_ref[2])
    sden = e0 + e1 + e2
    p = (p1_ref[...] * e0 + p2_ref[...] * e1 + p3_ref[...] * e2) / sden
    acc = (jnp.dot(p, w1p_ref[...], preferred_element_type=jnp.float32)
           + jnp.dot(xd, w1x_ref[...], preferred_element_type=jnp.float32)
           + jnp.dot(tgt_ref[...], w1t_ref[...], preferred_element_type=jnp.float32)
           + b1_ref[...])
    acc = jnp.maximum(acc, 0.0)
    acc = jnp.maximum(jnp.dot(acc, w2_ref[...], preferred_element_type=jnp.float32) + b2_ref[...], 0.0)
    acc = jnp.maximum(jnp.dot(acc, w3w_ref[...], preferred_element_type=jnp.float32) + b3_ref[...], 0.0)
    acc = jnp.maximum(jnp.dot(acc, w4_ref[...], preferred_element_type=jnp.float32) + b4_ref[...], 0.0)
    o_ref[...] = jnp.dot(acc, wo_ref[...], preferred_element_type=jnp.float32) + bo_ref[...]


def _tail(lo, pool, p1, p2, p3, w3, tgt,
          fc1_W, fc1_b, fc2_W, fc2_b, fc3_W, fc3_b, fc4_W, fc4_b,
          out_W, out_b):
    w1p = fc1_W[:128]
    w1x = fc1_W[128:206]
    w1t = fc1_W[206:]
    full = lambda *shape: pl.BlockSpec(shape, lambda: tuple(0 for _ in shape))
    args = (lo, pool, p1, p2, p3, w3, tgt, w1p, w1x, w1t,
            fc1_b.reshape(1, -1), fc2_W, fc2_b.reshape(1, -1),
            fc3_W, fc3_b.reshape(1, -1), fc4_W, fc4_b.reshape(1, -1),
            out_W, out_b.reshape(1, -1))
    in_specs = []
    for a in args:
        if a is w3:
            in_specs.append(pl.BlockSpec(memory_space=pltpu.SMEM))
        else:
            in_specs.append(full(*a.shape))
    return pl.pallas_call(
        _tail_kernel,
        in_specs=in_specs,
        out_specs=full(B_GRAPHS, 1),
        out_shape=jax.ShapeDtypeStruct((B_GRAPHS, 1), jnp.float32),
    )(*args)


def _seg_sum(data, segs, num):
    return jax.ops.segment_sum(data, segs, num_segments=num)


def _seg_max(data, segs, num):
    return jax.ops.segment_max(data, segs, num_segments=num)


def kernel(x, edge_index, batch, smile_feature, target,
           gat_W, gat_as, gat_ad, gat_b,
           gcn1_W, gcn1_b, fcg11_W, fcg11_b, fcg21_W, fcg21_b,
           gcn2_W, gcn2_b, fcg12_W, fcg12_b, fcg22_W, fcg22_b,
           gcn3_W, gcn3_b, fcg13_W, fcg13_b, fcg23_W, fcg23_b,
           fus_W, fus_b, w3, emb,
           blf_Wih, blf_Whh, blf_bih, blf_bhh,
           blr_Wih, blr_Whh, blr_bih, blr_bhh,
           l_Wih, l_Whh, l_bih, l_bhh,
           fc1_W, fc1_b, fc2_W, fc2_b, fc3_W, fc3_b, fc4_W, fc4_b,
           out_W, out_b):
    N = x.shape[0]
    loop = jnp.arange(N, dtype=jnp.int32)
    src_all = jnp.concatenate([edge_index[0].astype(jnp.int32), loop])
    dst_all = jnp.concatenate([edge_index[1].astype(jnp.int32), loop])
    perm = jnp.argsort(dst_all)
    srcs = src_all[perm]
    dsts = dst_all[perm]

    gat_W3 = gat_W.reshape(DF, HEADS, DF)
    Wa_src = jnp.einsum('dhf,hf->dh', gat_W3, gat_as)
    Wa_dst = jnp.einsum('dhf,hf->dh', gat_W3, gat_ad)
    Wa_src16 = jnp.pad(Wa_src, ((0, 0), (0, 6)))
    Wa_dst16 = jnp.pad(Wa_dst, ((0, 0), (0, 6)))
    Wcomb = jnp.concatenate([_pad_cols(gat_W), Wa_src16, Wa_dst16], axis=1)

    xp = jnp.pad(x, ((0, N_PAD - N), (0, 0)))
    hcomb = _mm(xp, Wcomb)
    h800 = hcomb[:, :F]
    asrc16 = hcomb[:, F:F + 16]
    adst16 = hcomb[:, F + 16:F + 32]

    alpha = jax.nn.leaky_relu(asrc16[srcs] + adst16[dsts], 0.2)
    amax = _seg_max(alpha, dsts, N_PAD)
    amax = jnp.where(jnp.isfinite(amax), amax, 0.0)
    e = jnp.exp(alpha - amax[dsts])
    den = _seg_sum(e, dsts, N_PAD)
    deg = den[:, 10]
    dis = jnp.where(deg > 0, deg ** -0.5, 0.0)
    e10 = e[:, :HEADS]
    rep = jnp.repeat(e10, HP, axis=1)
    agg = _seg_sum(h800[srcs] * rep, dsts, N_PAD)
    den_rep = jnp.repeat(den[:, :HEADS], HP, axis=1)
    x0 = _gat_finish(agg, den_rep, _pad_cols(gat_b))

    def gcn(r, W, b, resid):
        hw = _mm(r, _pad_both(W), row_scale=dis)
        aggv = _seg_sum(hw[srcs], dsts, N_PAD)
        return _scale_bias_relu_add(aggv, dis, _pad_cols(b), resid=resid)

    g1 = gcn(x0, gcn1_W, gcn1_b, None)
    r1 = g1 + x0
    g2 = gcn(r1, gcn2_W, gcn2_b, None)
    r2 = g2 + g1
    g3 = gcn(r2, gcn3_W, gcn3_b, None)
    r3 = g3 + g2
    fusion = _mm(r3, _pad_both(fus_W), bias=_pad_cols(fus_b), act="relu")

    batchp = jnp.concatenate([batch.astype(jnp.int32),
                              jnp.full((N_PAD - N,), B_GRAPHS, jnp.int32)])
    ones = jnp.ones((N_PAD, 1), jnp.float32)
    cnt = _seg_sum(ones, batchp, B_GRAPHS + 1)[:B_GRAPHS]

    def head(r, fcg1_W, fcg1_b, fcg2_W, fcg2_b):
        mx = _seg_max(r, batchp, B_GRAPHS + 1)[:B_GRAPHS]
        mx = jnp.where(jnp.isfinite(mx), mx, 0.0)
        sm = _seg_sum(r, batchp, B_GRAPHS + 1)[:B_GRAPHS]
        av = sm / jnp.maximum(cnt, 1.0)
        cat = jnp.concatenate([mx, av], axis=1)
        W1 = jnp.concatenate([_pad_rows(fcg1_W[:780]),
                              _pad_rows(fcg1_W[780:])], axis=0)
        p = _mm(cat, W1, bias=fcg1_b, act="relu", bm=128)
        return _mm(p, fcg2_W, bias=fcg2_b, bm=128)

    p1 = head(r1, fcg11_W, fcg11_b, fcg21_W, fcg21_b)
    p2 = head(r2, fcg12_W, fcg12_b, fcg22_W, fcg22_b)
    p3 = head(fusion, fcg13_W, fcg13_b, fcg23_W, fcg23_b)

    smile_pad = jnp.pad(smile_feature.astype(jnp.int32),
                        ((0, 0), (0, R_PAD - SEQ_L)))
    embx = _emb_lookup(smile_pad.reshape(-1), emb)
    wih1, whh1, b1v = _gate_pad(blf_Wih, blf_Whh, blf_bih, blf_bhh, 128)
    wih1r, whh1r, b1vr = _gate_pad(blr_Wih, blr_Whh, blr_bih, blr_bhh, 128)
    gxf = _mm(embx, wih1, bias=b1v).reshape(B_GRAPHS, R_PAD, 512)
    gxr = _mm(embx, wih1r, bias=b1vr).reshape(B_GRAPHS, R_PAD, 512)
    gx_stack = jnp.concatenate([gxf, jnp.flip(gxr, axis=0)], axis=1)
    whh_stack = jnp.concatenate([whh1, whh1r], axis=0)
    hs = _lstm_rec(gx_stack, whh_stack, 128, dual_half=R_PAD)
    bf = hs[:, :R_PAD, :64]
    bb = jnp.flip(hs[:, R_PAD:, :64], axis=0)
    l2in = jnp.concatenate([bf, bb], axis=-1)
    wih3, whh3, b3v = _gate_pad(l_Wih, l_Whh, l_bih, l_bhh, 128)
    gx3 = _mm(l2in.reshape(-1, 128), wih3, bias=b3v).reshape(B_GRAPHS, R_PAD, 512)
    lo = _lstm_rec(gx3, whh3, 128, mask_rows=SEQ_L)

    pool = jnp.asarray(_POOL_NP)
    return _tail(lo, pool, p1, p2, p3, w3, target,
                 fc1_W, fc1_b, fc2_W, fc2_b, fc3_W, fc3_b, fc4_W, fc4_b,
                 out_W, out_b)


# speedup vs baseline: 3.5867x; 1.8836x over previous
import functools

import numpy as np
import jax
import jax.numpy as jnp
from jax import lax
from jax.experimental import pallas as pl
from jax.experimental.pallas import tpu as pltpu
from jax.experimental.pallas import tpu_sc as plsc

N_NODES, N_EDGES, B_GRAPHS, SEQ_L = 10000, 160000, 128, 100
HEADS, DF = 10, 78
HP = 80
NP_SC = 7
F = NP_SC * 128
N_PAD = 10240
R_PAD = 104

NC, NS, L = 2, 16, 16
HALFN = N_PAD // NC
SUB_ROWS = HALFN // NS
ACC_ROWS = HALFN + L
TRASH = HALFN
SB = 128

_POS780 = np.arange(780)
_POS780 = (_POS780 // DF) * HP + (_POS780 % DF)


def _pool_mat_np(in_w=128, out_w=78):
    P = np.zeros((in_w, out_w), np.float32)
    for i in range(out_w):
        s = (i * in_w) // out_w
        e = int(np.ceil((i + 1) * in_w / out_w))
        P[s:e, i] = 1.0 / (e - s)
    return P


_POOL_NP = _pool_mat_np(128, 78)


def _pad_cols(W):
    out = jnp.zeros(W.shape[:-1] + (F,), jnp.float32)
    return out.at[..., _POS780].set(W)


def _pad_rows(W):
    out = jnp.zeros((F,) + W.shape[1:], jnp.float32)
    return out.at[_POS780].set(W)


def _pad_both(W):
    return _pad_cols(_pad_rows(W))


def _mm_kernel(a_ref, b_ref, bias_ref, s_ref, o_ref, *, act, scale):
    acc = jnp.dot(a_ref[...], b_ref[...], preferred_element_type=jnp.float32)
    acc = acc + bias_ref[...]
    if scale:
        acc = acc * s_ref[...]
    if act == "relu":
        acc = jnp.maximum(acc, 0.0)
    o_ref[...] = acc


def _mm(A, B, bias=None, act=None, row_scale=None, bm=1024):
    M, K = A.shape
    N = B.shape[1]
    Mp = ((M + bm - 1) // bm) * bm
    if Mp != M:
        A = jnp.pad(A, ((0, Mp - M), (0, 0)))
    if bias is None:
        bias = jnp.zeros((N,), jnp.float32)
    bias2 = bias.reshape(1, N)
    scale = row_scale is not None
    if scale:
        s = row_scale.reshape(M, 1)
        if Mp != M:
            s = jnp.pad(s, ((0, Mp - M), (0, 0)))
    else:
        s = jnp.zeros((Mp, 1), jnp.float32)
    out = pl.pallas_call(
        functools.partial(_mm_kernel, act=act, scale=scale),
        grid=(Mp // bm,),
        in_specs=[
            pl.BlockSpec((bm, K), lambda i: (i, 0)),
            pl.BlockSpec((K, N), lambda i: (0, 0)),
            pl.BlockSpec((1, N), lambda i: (0, 0)),
            pl.BlockSpec((bm, 1), lambda i: (i, 0)),
        ],
        out_specs=pl.BlockSpec((bm, N), lambda i: (i, 0)),
        out_shape=jax.ShapeDtypeStruct((Mp, N), jnp.float32),
    )(A, B, bias2, s)
    return out[:M] if Mp != M else out


def _gat_finish_kernel(agg_ref, den_ref, b_ref, o_ref):
    den = den_ref[...]
    acc = agg_ref[...]
    parts = []
    for h in range(HEADS):
        d = den[:, h * HP:h * HP + 1]
        parts.append(acc[:, h * HP:(h + 1) * HP] / (d + 1e-16))
    parts.append(acc[:, HEADS * HP:])
    o_ref[...] = jnp.maximum(jnp.concatenate(parts, axis=1) + b_ref[...], 0.0)


def _gat_finish(agg, den_rep, bias, bm=1024):
    M = agg.shape[0]
    return pl.pallas_call(
        _gat_finish_kernel,
        grid=(M // bm,),
        in_specs=[
            pl.BlockSpec((bm, F), lambda i: (i, 0)),
            pl.BlockSpec((bm, F), lambda i: (i, 0)),
            pl.BlockSpec((1, F), lambda i: (0, 0)),
        ],
        out_specs=pl.BlockSpec((bm, F), lambda i: (i, 0)),
        out_shape=jax.ShapeDtypeStruct((M, F), jnp.float32),
    )(agg, den_rep, bias.reshape(1, F))


def _scale_bias_relu_add_kernel(a_ref, s_ref, b_ref, c_ref, o_ref, *, resid):
    v = jnp.maximum(a_ref[...] * s_ref[...] + b_ref[...], 0.0)
    if resid:
        v = v + c_ref[...]
    o_ref[...] = v


def _scale_bias_relu_add(a, row_scale, bias, resid=None, bm=1024):
    M, N = a.shape
    s = row_scale.reshape(M, 1)
    has_resid = resid is not None
    c = resid if has_resid else jnp.zeros((M, 1), jnp.float32)
    cN = c.shape[1]
    return pl.pallas_call(
        functools.partial(_scale_bias_relu_add_kernel, resid=has_resid),
        grid=(M // bm,),
        in_specs=[
            pl.BlockSpec((bm, N), lambda i: (i, 0)),
            pl.BlockSpec((bm, 1), lambda i: (i, 0)),
            pl.BlockSpec((1, N), lambda i: (0, 0)),
            pl.BlockSpec((bm, cN), lambda i: (i, 0)),
        ],
        out_specs=pl.BlockSpec((bm, N), lambda i: (i, 0)),
        out_shape=jax.ShapeDtypeStruct((M, N), jnp.float32),
    )(a, s, bias.reshape(1, N), c)


def _emb_kernel(idx_ref, emb_ref, o_ref):
    idx = idx_ref[...]
    iot = lax.broadcasted_iota(jnp.int32, (1, emb_ref.shape[0]), 1)
    onehot = (idx == iot).astype(jnp.float32)
    o_ref[...] = jnp.dot(onehot, emb_ref[...], preferred_element_type=jnp.float32)


def _emb_lookup(idx_flat, emb, bm=1024):
    M = idx_flat.shape[0]
    V, D = emb.shape
    return pl.pallas_call(
        _emb_kernel,
        grid=(M // bm,),
        in_specs=[
            pl.BlockSpec((bm, 1), lambda i: (i, 0)),
            pl.BlockSpec((V, D), lambda i: (0, 0)),
        ],
        out_specs=pl.BlockSpec((bm, D), lambda i: (i, 0)),
        out_shape=jax.ShapeDtypeStruct((M, D), jnp.float32),
    )(idx_flat.reshape(M, 1).astype(jnp.int32), emb)


def _lstm_rec_kernel(gx_ref, whh_ref, o_ref, h_ref, c_ref, *, Hg, mask_rows,
                     dual_half):
    t = pl.program_id(0)

    @pl.when(t == 0)
    def _():
        h_ref[...] = jnp.zeros_like(h_ref)
        c_ref[...] = jnp.zeros_like(c_ref)

    g = gx_ref[0] + jnp.dot(h_ref[...], whh_ref[...],
                            preferred_element_type=jnp.float32)
    i = g[:, 0 * Hg:1 * Hg]
    f = g[:, 1 * Hg:2 * Hg]
    gg = g[:, 2 * Hg:3 * Hg]
    o = g[:, 3 * Hg:4 * Hg]
    c = jax.nn.sigmoid(f) * c_ref[...] + jax.nn.sigmoid(i) * jnp.tanh(gg)
    h = jax.nn.sigmoid(o) * jnp.tanh(c)
    c_ref[...] = c
    if dual_half is None:
        h_ref[...] = h
    else:
        rows = lax.broadcasted_iota(jnp.int32, h.shape, 0)
        m = (rows < dual_half).astype(jnp.float32)
        h_ref[...] = jnp.concatenate([h * m, h * (1.0 - m)], axis=1)
    if mask_rows is not None:
        rows = lax.broadcasted_iota(jnp.int32, h.shape, 0)
        h = jnp.where(rows < mask_rows, h, 0.0)
    o_ref[0] = h


def _lstm_rec(gx, whh_t, Hg, mask_rows=None, dual_half=None):
    T, R, G = gx.shape
    Hw = whh_t.shape[0]
    return pl.pallas_call(
        functools.partial(_lstm_rec_kernel, Hg=Hg, mask_rows=mask_rows,
                          dual_half=dual_half),
        grid=(T,),
        in_specs=[
            pl.BlockSpec((1, R, G), lambda t: (t, 0, 0)),
            pl.BlockSpec((Hw, G), lambda t: (0, 0)),
        ],
        out_specs=pl.BlockSpec((1, R, Hg), lambda t: (t, 0, 0)),
        out_shape=jax.ShapeDtypeStruct((T, R, Hg), jnp.float32),
        scratch_shapes=[
            pltpu.VMEM((R, Hw), jnp.float32),
            pltpu.VMEM((R, Hg), jnp.float32),
        ],
    )(gx, whh_t)


def _gate_pad(Wih, Whh, bih, bhh, Hg):
    H4, K = Wih.shape
    H = H4 // 4
    Hin = Whh.shape[1]
    wih = jnp.zeros((K, 4 * Hg), jnp.float32)
    whh = jnp.zeros((Hg, 4 * Hg), jnp.float32)
    b = jnp.zeros((4 * Hg,), jnp.float32)
    for g in range(4):
        wih = wih.at[:, g * Hg:g * Hg + H].set(Wih[g * H:(g + 1) * H].T)
        whh = whh.at[:Hin, g * Hg:g * Hg + H].set(Whh[g * H:(g + 1) * H].T)
        b = b.at[g * Hg:g * Hg + H].set(bih[g * H:(g + 1) * H] +
                                        bhh[g * H:(g + 1) * H])
    return wih, whh, b


def _tail_kernel(lo_ref, pool_ref, p1_ref, p2_ref, p3_ref, w3_ref, tgt_ref,
                 w1p_ref, w1x_ref, w1t_ref, b1_ref, w2_ref, b2_ref,
                 w3w_ref, b3_ref, w4_ref, b4_ref, wo_ref, bo_ref, o_ref):
    losum = jnp.sum(lo_ref[...], axis=1) * (1.0 / SEQ_L)
    xd = jnp.dot(losum, pool_ref[...], preferred_element_type=jnp.float32)
    e0 = jnp.exp(w3_ref[0])
    e1 = jnp.exp(w3_ref[1])
    e2 = jnp.exp(w3_ref[2])
    sden = e0 + e1 + e2
    p = (p1_ref[...] * e0 + p2_ref[...] * e1 + p3_ref[...] * e2) / sden
    acc = (jnp.dot(p, w1p_ref[...], preferred_element_type=jnp.float32)
           + jnp.dot(xd, w1x_ref[...], preferred_element_type=jnp.float32)
           + jnp.dot(tgt_ref[...], w1t_ref[...], preferred_element_type=jnp.float32)
           + b1_ref[...])
    acc = jnp.maximum(acc, 0.0)
    acc = jnp.maximum(jnp.dot(acc, w2_ref[...], preferred_element_type=jnp.float32) + b2_ref[...], 0.0)
    acc = jnp.maximum(jnp.dot(acc, w3w_ref[...], preferred_element_type=jnp.float32) + b3_ref[...], 0.0)
    acc = jnp.maximum(jnp.dot(acc, w4_ref[...], preferred_element_type=jnp.float32) + b4_ref[...], 0.0)
    o_ref[...] = jnp.dot(acc, wo_ref[...], preferred_element_type=jnp.float32) + bo_ref[...]


def _tail(lo, pool, p1, p2, p3, w3, tgt,
          fc1_W, fc1_b, fc2_W, fc2_b, fc3_W, fc3_b, fc4_W, fc4_b,
          out_W, out_b):
    w1p = fc1_W[:128]
    w1x = fc1_W[128:206]
    w1t = fc1_W[206:]
    full = lambda *shape: pl.BlockSpec(shape, lambda: tuple(0 for _ in shape))
    args = (lo, pool, p1, p2, p3, w3, tgt, w1p, w1x, w1t,
            fc1_b.reshape(1, -1), fc2_W, fc2_b.reshape(1, -1),
            fc3_W, fc3_b.reshape(1, -1), fc4_W, fc4_b.reshape(1, -1),
            out_W, out_b.reshape(1, -1))
    in_specs = []
    for a in args:
        if a is w3:
            in_specs.append(pl.BlockSpec(memory_space=pltpu.SMEM))
        else:
            in_specs.append(full(*a.shape))
    return pl.pallas_call(
        _tail_kernel,
        in_specs=in_specs,
        out_specs=full(B_GRAPHS, 1),
        out_shape=jax.ShapeDtypeStruct((B_GRAPHS, 1), jnp.float32),
    )(*args)


def _seg_sum(data, segs, num):
    return jax.ops.segment_sum(data, segs, num_segments=num)


def _seg_max(data, segs, num):
    return jax.ops.segment_max(data, segs, num_segments=num)


_SC_MESH = plsc.VectorSubcoreMesh(core_axis_name="c", subcore_axis_name="s")


def _sc_agg_kernel(hw7, srcs, dsts, meta, out, acc, mvec, zbuf, sidx, didx,
                   gidx, loc, rows, sem):
    c = lax.axis_index("c")
    s = lax.axis_index("s")
    w = c * NS + s
    nodebase = c * HALFN

    def zrow_body(r, _):
        for j in range(8):
            zbuf[r, pl.ds(j * L, L)] = jnp.zeros((L,), jnp.float32)
        return 0

    lax.fori_loop(0, SB, zrow_body, 0)
    pltpu.sync_copy(meta, mvec)
    mrow = mvec[w]
    bstart = mrow[0]
    bcount = mrow[1]

    for p in range(NP_SC):
        base_r = s * SUB_ROWS
        pltpu.sync_copy(zbuf, acc.at[pl.ds(base_r, SB)])
        pltpu.sync_copy(zbuf, acc.at[pl.ds(base_r + SB, SB)])
        pltpu.sync_copy(zbuf.at[pl.ds(0, SUB_ROWS - 2 * SB)],
                        acc.at[pl.ds(base_r + 2 * SB, SUB_ROWS - 2 * SB)])
        plsc.subcore_barrier()

        def batch_body(k, _):
            base = (bstart + k) * SB
            pltpu.sync_copy(srcs.at[pl.ds(base, SB)], sidx)
            pltpu.sync_copy(dsts.at[pl.ds(base, SB)], didx)
            for j in range(SB // L):
                s16 = sidx[pl.ds(j * L, L)]
                d16 = didx[pl.ds(j * L, L)]
                gidx[pl.ds(j * L, L)] = s16 * NP_SC + p
                inh = (d16 >= nodebase) & (d16 < nodebase + HALFN)
                loc[pl.ds(j * L, L)] = jnp.where(inh, d16 - nodebase, TRASH)
            pltpu.async_copy(hw7.at[gidx], rows, sem).wait()
            pltpu.sync_copy(rows, acc.at[loc], add=True)
            return 0

        lax.fori_loop(0, bcount, batch_body, 0)
        plsc.subcore_barrier()
        pltpu.sync_copy(
            acc.at[pl.ds(base_r, SUB_ROWS)],
            out.at[pl.ds(p * N_PAD + nodebase + base_r, SUB_ROWS)])
        plsc.subcore_barrier()


def _sc_agg(hw, srcs_p, dsts_p, meta):
    hw7 = hw.reshape(N_PAD * NP_SC, 128)
    out = pl.kernel(
        _sc_agg_kernel,
        out_type=jax.ShapeDtypeStruct((NP_SC * N_PAD, 128), jnp.float32),
        mesh=_SC_MESH,
        scratch_types=[
            pltpu.VMEM_SHARED((ACC_ROWS, 128), jnp.float32),
            pltpu.VMEM((NC * NS, L), jnp.int32),
            pltpu.VMEM((SB, 128), jnp.float32),
            pltpu.VMEM((SB,), jnp.int32),
            pltpu.VMEM((SB,), jnp.int32),
            pltpu.VMEM((SB,), jnp.int32),
            pltpu.VMEM((SB,), jnp.int32),
            pltpu.VMEM((SB, 128), jnp.float32),
            pltpu.SemaphoreType.DMA,
        ],
    )(hw7, srcs_p, dsts_p, meta)
    return out.reshape(NP_SC, N_PAD, 128).transpose(1, 0, 2).reshape(N_PAD, F)


def _edge_meta(srcs, dsts):
    E = dsts.shape[0]
    EP = -(-E // SB) * SB
    srcs_p = jnp.full((EP,), N_PAD - 1, jnp.int32).at[:E].set(srcs)
    dsts_p = jnp.full((EP,), jnp.int32(2**30)).at[:E].set(dsts)
    e_mid = jnp.searchsorted(dsts, HALFN).astype(jnp.int32)
    b0c = jnp.stack([jnp.int32(0), e_mid // SB])
    b1c = jnp.stack([-(-e_mid // SB), jnp.int32(-(-E // SB))])
    nb = b1c - b0c
    per = -(-nb // NS)
    s0 = jnp.minimum(b0c[:, None] + jnp.arange(NS, dtype=jnp.int32) *
                     per[:, None], b1c[:, None])
    s1 = jnp.minimum(s0 + per[:, None], b1c[:, None])
    meta = jnp.zeros((NC * NS, L), jnp.int32)
    meta = meta.at[:, 0].set(s0.reshape(-1))
    meta = meta.at[:, 1].set((s1 - s0).reshape(-1))
    return srcs_p, dsts_p, meta


def kernel(x, edge_index, batch, smile_feature, target,
           gat_W, gat_as, gat_ad, gat_b,
           gcn1_W, gcn1_b, fcg11_W, fcg11_b, fcg21_W, fcg21_b,
           gcn2_W, gcn2_b, fcg12_W, fcg12_b, fcg22_W, fcg22_b,
           gcn3_W, gcn3_b, fcg13_W, fcg13_b, fcg23_W, fcg23_b,
           fus_W, fus_b, w3, emb,
           blf_Wih, blf_Whh, blf_bih, blf_bhh,
           blr_Wih, blr_Whh, blr_bih, blr_bhh,
           l_Wih, l_Whh, l_bih, l_bhh,
           fc1_W, fc1_b, fc2_W, fc2_b, fc3_W, fc3_b, fc4_W, fc4_b,
           out_W, out_b):
    N = x.shape[0]
    loop = jnp.arange(N, dtype=jnp.int32)
    src_all = jnp.concatenate([edge_index[0].astype(jnp.int32), loop])
    dst_all = jnp.concatenate([edge_index[1].astype(jnp.int32), loop])
    perm = jnp.argsort(dst_all)
    srcs = src_all[perm]
    dsts = dst_all[perm]
    srcs_p, dsts_p, emeta = _edge_meta(srcs, dsts)

    gat_W3 = gat_W.reshape(DF, HEADS, DF)
    Wa_src = jnp.einsum('dhf,hf->dh', gat_W3, gat_as)
    Wa_dst = jnp.einsum('dhf,hf->dh', gat_W3, gat_ad)
    Wa_src16 = jnp.pad(Wa_src, ((0, 0), (0, 6)))
    Wa_dst16 = jnp.pad(Wa_dst, ((0, 0), (0, 6)))
    Wcomb = jnp.concatenate([_pad_cols(gat_W), Wa_src16, Wa_dst16], axis=1)

    xp = jnp.pad(x, ((0, N_PAD - N), (0, 0)))
    hcomb = _mm(xp, Wcomb)
    h800 = hcomb[:, :F]
    asrc16 = hcomb[:, F:F + 16]
    adst16 = hcomb[:, F + 16:F + 32]

    alpha = jax.nn.leaky_relu(asrc16[srcs] + adst16[dsts], 0.2)
    amax = _seg_max(alpha, dsts, N_PAD)
    amax = jnp.where(jnp.isfinite(amax), amax, 0.0)
    e = jnp.exp(alpha - amax[dsts])
    den = _seg_sum(e, dsts, N_PAD)
    deg = den[:, 10]
    dis = jnp.where(deg > 0, deg ** -0.5, 0.0)
    e10 = e[:, :HEADS]
    rep = jnp.pad(jnp.repeat(e10, HP, axis=1), ((0, 0), (0, F - HEADS * HP)))
    agg = _seg_sum(h800[srcs] * rep, dsts, N_PAD)
    den_rep = jnp.pad(jnp.repeat(den[:, :HEADS], HP, axis=1),
                      ((0, 0), (0, F - HEADS * HP)))
    x0 = _gat_finish(agg, den_rep, _pad_cols(gat_b))

    def gcn(r, W, b, resid):
        hw = _mm(r, _pad_both(W), row_scale=dis)
        aggv = _sc_agg(hw, srcs_p, dsts_p, emeta)
        return _scale_bias_relu_add(aggv, dis, _pad_cols(b), resid=resid)

    g1 = gcn(x0, gcn1_W, gcn1_b, None)
    r1 = g1 + x0
    g2 = gcn(r1, gcn2_W, gcn2_b, None)
    r2 = g2 + g1
    g3 = gcn(r2, gcn3_W, gcn3_b, None)
    r3 = g3 + g2
    fusion = _mm(r3, _pad_both(fus_W), bias=_pad_cols(fus_b), act="relu")

    batchp = jnp.concatenate([batch.astype(jnp.int32),
                              jnp.full((N_PAD - N,), B_GRAPHS, jnp.int32)])
    ones = jnp.ones((N_PAD, 1), jnp.float32)
    cnt = _seg_sum(ones, batchp, B_GRAPHS + 1)[:B_GRAPHS]

    def head(r, fcg1_W, fcg1_b, fcg2_W, fcg2_b):
        mx = _seg_max(r, batchp, B_GRAPHS + 1)[:B_GRAPHS]
        mx = jnp.where(jnp.isfinite(mx), mx, 0.0)
        sm = _seg_sum(r, batchp, B_GRAPHS + 1)[:B_GRAPHS]
        av = sm / jnp.maximum(cnt, 1.0)
        cat = jnp.concatenate([mx, av], axis=1)
        W1 = jnp.concatenate([_pad_rows(fcg1_W[:780]),
                              _pad_rows(fcg1_W[780:])], axis=0)
        p = _mm(cat, W1, bias=fcg1_b, act="relu", bm=128)
        return _mm(p, fcg2_W, bias=fcg2_b, bm=128)

    p1 = head(r1, fcg11_W, fcg11_b, fcg21_W, fcg21_b)
    p2 = head(r2, fcg12_W, fcg12_b, fcg22_W, fcg22_b)
    p3 = head(fusion, fcg13_W, fcg13_b, fcg23_W, fcg23_b)

    smile_pad = jnp.pad(smile_feature.astype(jnp.int32),
                        ((0, 0), (0, R_PAD - SEQ_L)))
    embx = _emb_lookup(smile_pad.reshape(-1), emb)
    wih1, whh1, b1v = _gate_pad(blf_Wih, blf_Whh, blf_bih, blf_bhh, 128)
    wih1r, whh1r, b1vr = _gate_pad(blr_Wih, blr_Whh, blr_bih, blr_bhh, 128)
    gxf = _mm(embx, wih1, bias=b1v).reshape(B_GRAPHS, R_PAD, 512)
    gxr = _mm(embx, wih1r, bias=b1vr).reshape(B_GRAPHS, R_PAD, 512)
    gx_stack = jnp.concatenate([gxf, jnp.flip(gxr, axis=0)], axis=1)
    whh_stack = jnp.concatenate([whh1, whh1r], axis=0)
    hs = _lstm_rec(gx_stack, whh_stack, 128, dual_half=R_PAD)
    bf = hs[:, :R_PAD, :64]
    bb = jnp.flip(hs[:, R_PAD:, :64], axis=0)
    l2in = jnp.concatenate([bf, bb], axis=-1)
    wih3, whh3, b3v = _gate_pad(l_Wih, l_Whh, l_bih, l_bhh, 128)
    gx3 = _mm(l2in.reshape(-1, 128), wih3, bias=b3v).reshape(B_GRAPHS, R_PAD, 512)
    lo = _lstm_rec(gx3, whh3, 128, mask_rows=SEQ_L)

    pool = jnp.asarray(_POOL_NP)
    return _tail(lo, pool, p1, p2, p3, w3, target,
                 fc1_W, fc1_b, fc2_W, fc2_b, fc3_W, fc3_b, fc4_W, fc4_b,
                 out_W, out_b)
